# initial kernel scaffold (unmeasured)
import jax
import jax.numpy as jnp
from jax import lax
from jax.experimental import pallas as pl
from jax.experimental.pallas import tpu as pltpu

N_DEV = 4


def kernel(x, w_mat, scale_x, scale_w):
    m_per, k = x.shape
    n = w_mat.shape[1]

    def body(x_ref, w_ref, sx_ref, sw_ref, out_ref,
             comm_ref, w_bf_ref, send_sems, recv_sems):
        my = lax.axis_index("i")
        left = (my - 1) % N_DEV
        right = (my + 1) % N_DEV

        barrier_sem = pltpu.get_barrier_semaphore()
        for nbr in [left, right]:
            pl.semaphore_signal(
                barrier_sem, inc=1,
                device_id=(nbr,), device_id_type=pl.DeviceIdType.MESH,
            )
        pl.semaphore_wait(barrier_sem, 2)

        s = sx_ref[0] * sw_ref[0]
        w_bf_ref[...] = w_ref[...].astype(jnp.bfloat16)

        def gemm_store(chunk_fp8, origin):
            acc = jnp.dot(
                chunk_fp8.astype(jnp.bfloat16), w_bf_ref[...],
                preferred_element_type=jnp.float32,
            )
            y = acc * s
            z = jnp.clip(y, -60.0, 60.0)
            out_ref[pl.ds(origin * m_per, m_per), :] = y / (1.0 + jnp.exp(-z))

        comm_ref[0, :, :] = x_ref[...].astype(jnp.float8_e4m3fn)
        gemm_store(comm_ref[0, :, :], my)

        for h in range(N_DEV - 1):
            rdma = pltpu.make_async_remote_copy(
                src_ref=comm_ref.at[h],
                dst_ref=comm_ref.at[h + 1],
                send_sem=send_sems.at[h],
                recv_sem=recv_sems.at[h],
                device_id=(right,),
                device_id_type=pl.DeviceIdType.MESH,
            )
            rdma.start()
            rdma.wait()
            origin = (my - h - 1) % N_DEV
            gemm_store(comm_ref[h + 1, :, :], origin)

    return pl.pallas_call(
        body,
        out_shape=jax.ShapeDtypeStruct((N_DEV * m_per, n), jnp.float32),
        in_specs=[
            pl.BlockSpec(memory_space=pltpu.VMEM),
            pl.BlockSpec(memory_space=pltpu.VMEM),
            pl.BlockSpec(memory_space=pltpu.SMEM),
            pl.BlockSpec(memory_space=pltpu.SMEM),
        ],
        out_specs=pl.BlockSpec(memory_space=pltpu.VMEM),
        scratch_shapes=[
            pltpu.VMEM((N_DEV, m_per, k), jnp.float8_e4m3fn),
            pltpu.VMEM((k, n), jnp.bfloat16),
            pltpu.SemaphoreType.DMA((N_DEV - 1,)),
            pltpu.SemaphoreType.DMA((N_DEV - 1,)),
        ],
        compiler_params=pltpu.CompilerParams(collective_id=0),
    )(x, w_mat, scale_x, scale_w)


# baseline (device time: 181630 ns/iter reference)
import jax
import jax.numpy as jnp
from jax import lax
from jax.experimental import pallas as pl
from jax.experimental.pallas import tpu as pltpu

N_DEV = 4


def kernel(x, w_mat, scale_x, scale_w):
    m_per, k = x.shape
    n = w_mat.shape[1]

    def body(x_ref, w_ref, sx_ref, sw_ref, out_ref,
             comm_ref, w_bf_ref, send_sems, recv_sems):
        my = lax.axis_index("i")
        left = (my - 1) % N_DEV
        right = (my + 1) % N_DEV

        barrier_sem = pltpu.get_barrier_semaphore()
        for nbr in [left, right]:
            pl.semaphore_signal(
                barrier_sem, inc=1,
                device_id=(nbr,), device_id_type=pl.DeviceIdType.MESH,
            )
        pl.semaphore_wait(barrier_sem, 2)

        s = sx_ref[0] * sw_ref[0]
        w_bf_ref[...] = w_ref[...].astype(jnp.bfloat16)

        def gemm_store(chunk_fp8, origin):
            acc = jnp.dot(
                chunk_fp8.astype(jnp.bfloat16), w_bf_ref[...],
                preferred_element_type=jnp.float32,
            )
            y = acc * s
            z = jnp.clip(y, -60.0, 60.0)
            out_ref[pl.ds(origin * m_per, m_per), :] = y / (1.0 + jnp.exp(-z))

        comm_ref[0, :, :] = x_ref[...].astype(jnp.float8_e4m3fn)
        gemm_store(comm_ref[0, :, :], my)

        for h in range(N_DEV - 1):
            rdma = pltpu.make_async_remote_copy(
                src_ref=comm_ref.at[h],
                dst_ref=comm_ref.at[h + 1],
                send_sem=send_sems.at[h],
                recv_sem=recv_sems.at[h],
                device_id=(right,),
                device_id_type=pl.DeviceIdType.MESH,
            )
            rdma.start()
            rdma.wait()
            origin = (my - h - 1) % N_DEV
            gemm_store(comm_ref[h + 1, :, :], origin)

    return pl.pallas_call(
        body,
        out_shape=jax.ShapeDtypeStruct((N_DEV * m_per, n), jnp.float32),
        in_specs=[
            pl.BlockSpec(memory_space=pltpu.VMEM),
            pl.BlockSpec(memory_space=pltpu.VMEM),
            pl.BlockSpec(memory_space=pltpu.SMEM),
            pl.BlockSpec(memory_space=pltpu.SMEM),
        ],
        out_specs=pl.BlockSpec(memory_space=pltpu.VMEM),
        scratch_shapes=[
            pltpu.VMEM((N_DEV, m_per, k), jnp.float8_e4m3fn),
            pltpu.VMEM((k, n), jnp.bfloat16),
            pltpu.SemaphoreType.DMA((N_DEV - 1,)),
            pltpu.SemaphoreType.DMA((N_DEV - 1,)),
        ],
        compiler_params=pltpu.CompilerParams(
            collective_id=0,
            vmem_limit_bytes=100 * 1024 * 1024,
        ),
    )(x, w_mat, scale_x, scale_w)


# device time: 99663 ns/iter; 1.8224x vs baseline; 1.8224x over previous
import jax
import jax.numpy as jnp
from jax import lax
from jax.experimental import pallas as pl
from jax.experimental.pallas import tpu as pltpu

N_DEV = 4


def kernel(x, w_mat, scale_x, scale_w):
    m_per, k = x.shape
    n = w_mat.shape[1]
    m_half = m_per // 2

    def body(x_ref, w_ref, sx_ref, sw_ref, out_ref,
             cw_ref, ccw_ref, w_bf_ref,
             send_cw, recv_cw, send_ccw, recv_ccw):
        my = lax.axis_index("i")
        left = (my - 1) % N_DEV
        right = (my + 1) % N_DEV

        barrier_sem = pltpu.get_barrier_semaphore()
        for nbr in [left, right]:
            pl.semaphore_signal(
                barrier_sem, inc=1,
                device_id=(nbr,), device_id_type=pl.DeviceIdType.MESH,
            )
        pl.semaphore_wait(barrier_sem, 2)

        s = sx_ref[0] * sw_ref[0]

        cw_ref[0, :, :] = x_ref[:m_half, :].astype(jnp.float8_e4m3fn)
        ccw_ref[0, :, :] = x_ref[m_half:, :].astype(jnp.float8_e4m3fn)

        def start_hop(h):
            r_cw = pltpu.make_async_remote_copy(
                src_ref=cw_ref.at[h - 1], dst_ref=cw_ref.at[h],
                send_sem=send_cw.at[h - 1], recv_sem=recv_cw.at[h - 1],
                device_id=(right,), device_id_type=pl.DeviceIdType.MESH,
            )
            r_ccw = pltpu.make_async_remote_copy(
                src_ref=ccw_ref.at[h - 1], dst_ref=ccw_ref.at[h],
                send_sem=send_ccw.at[h - 1], recv_sem=recv_ccw.at[h - 1],
                device_id=(left,), device_id_type=pl.DeviceIdType.MESH,
            )
            r_cw.start()
            r_ccw.start()
            return r_cw, r_ccw

        def gemm_store(chunk_fp8, row_start):
            acc = jnp.dot(
                chunk_fp8.astype(jnp.bfloat16), w_bf_ref[...],
                preferred_element_type=jnp.float32,
            )
            y = acc * s
            z = jnp.clip(y, -60.0, 60.0)
            out_ref[pl.ds(row_start, chunk_fp8.shape[0]), :] = (
                y / (1.0 + jnp.exp(-z))
            )

        rdmas = [start_hop(1)]
        w_bf_ref[...] = w_ref[...].astype(jnp.bfloat16)
        gemm_store(x_ref[...].astype(jnp.float8_e4m3fn), my * m_per)

        for h in range(1, N_DEV):
            r_cw, r_ccw = rdmas[h - 1]
            r_cw.wait_recv()
            r_ccw.wait_recv()
            if h + 1 < N_DEV:
                rdmas.append(start_hop(h + 1))
            origin_cw = (my - h) % N_DEV
            origin_ccw = (my + h) % N_DEV
            gemm_store(cw_ref[h, :, :], origin_cw * m_per)
            gemm_store(ccw_ref[h, :, :], origin_ccw * m_per + m_half)

        for r_cw, r_ccw in rdmas:
            r_cw.wait_send()
            r_ccw.wait_send()

    return pl.pallas_call(
        body,
        out_shape=jax.ShapeDtypeStruct((N_DEV * m_per, n), jnp.float32),
        in_specs=[
            pl.BlockSpec(memory_space=pltpu.VMEM),
            pl.BlockSpec(memory_space=pltpu.VMEM),
            pl.BlockSpec(memory_space=pltpu.SMEM),
            pl.BlockSpec(memory_space=pltpu.SMEM),
        ],
        out_specs=pl.BlockSpec(memory_space=pltpu.VMEM),
        scratch_shapes=[
            pltpu.VMEM((N_DEV, m_half, k), jnp.float8_e4m3fn),
            pltpu.VMEM((N_DEV, m_half, k), jnp.float8_e4m3fn),
            pltpu.VMEM((k, n), jnp.bfloat16),
            pltpu.SemaphoreType.DMA((N_DEV - 1,)),
            pltpu.SemaphoreType.DMA((N_DEV - 1,)),
            pltpu.SemaphoreType.DMA((N_DEV - 1,)),
            pltpu.SemaphoreType.DMA((N_DEV - 1,)),
        ],
        compiler_params=pltpu.CompilerParams(
            collective_id=0,
            vmem_limit_bytes=100 * 1024 * 1024,
        ),
    )(x, w_mat, scale_x, scale_w)


# device time: 96226 ns/iter; 1.8875x vs baseline; 1.0357x over previous
import jax
import jax.numpy as jnp
from jax import lax
from jax.experimental import pallas as pl
from jax.experimental.pallas import tpu as pltpu

N_DEV = 4
N_SUB = 2


def kernel(x, w_mat, scale_x, scale_w):
    m_per, k = x.shape
    n = w_mat.shape[1]
    m_half = m_per // 2
    k_sub = k // N_SUB

    def body(x_ref, w_ref, sx_ref, sw_ref, out_ref,
             cw_ref, ccw_ref, w_bf_ref,
             send_cw, recv_cw, send_ccw, recv_ccw):
        my = lax.axis_index("i")
        left = (my - 1) % N_DEV
        right = (my + 1) % N_DEV

        barrier_sem = pltpu.get_barrier_semaphore()
        for nbr in [left, right]:
            pl.semaphore_signal(
                barrier_sem, inc=1,
                device_id=(nbr,), device_id_type=pl.DeviceIdType.MESH,
            )
        pl.semaphore_wait(barrier_sem, 2)

        s = sx_ref[0] * sw_ref[0]

        for sub in range(N_SUB):
            cols = pl.ds(sub * k_sub, k_sub)
            cw_ref[0, sub, :, :] = x_ref[:m_half, cols].astype(jnp.float8_e4m3fn)
            ccw_ref[0, sub, :, :] = x_ref[m_half:, cols].astype(jnp.float8_e4m3fn)

        def start_hop(h, sub):
            r_cw = pltpu.make_async_remote_copy(
                src_ref=cw_ref.at[h - 1, sub], dst_ref=cw_ref.at[h, sub],
                send_sem=send_cw.at[h - 1, sub], recv_sem=recv_cw.at[h - 1, sub],
                device_id=(right,), device_id_type=pl.DeviceIdType.MESH,
            )
            r_ccw = pltpu.make_async_remote_copy(
                src_ref=ccw_ref.at[h - 1, sub], dst_ref=ccw_ref.at[h, sub],
                send_sem=send_ccw.at[h - 1, sub], recv_sem=recv_ccw.at[h - 1, sub],
                device_id=(left,), device_id_type=pl.DeviceIdType.MESH,
            )
            r_cw.start()
            r_ccw.start()
            return r_cw, r_ccw

        def dot_sub(chunk_fp8, sub):
            return jnp.dot(
                chunk_fp8.astype(jnp.bfloat16),
                w_bf_ref[pl.ds(sub * k_sub, k_sub), :],
                preferred_element_type=jnp.float32,
            )

        def epilogue_store(acc, row_start, rows):
            y = acc * s
            z = jnp.clip(y, -60.0, 60.0)
            out_ref[pl.ds(row_start, rows), :] = y / (1.0 + jnp.exp(-z))

        rdmas = {(1, sub): start_hop(1, sub) for sub in range(N_SUB)}
        w_bf_ref[...] = w_ref[...].astype(jnp.bfloat16)
        acc_local = jnp.dot(
            x_ref[...].astype(jnp.bfloat16), w_bf_ref[...],
            preferred_element_type=jnp.float32,
        )
        epilogue_store(acc_local, my * m_per, m_per)

        for h in range(1, N_DEV):
            acc_cw = None
            acc_ccw = None
            for sub in range(N_SUB):
                r_cw, r_ccw = rdmas[(h, sub)]
                r_cw.wait_recv()
                r_ccw.wait_recv()
                if h + 1 < N_DEV:
                    rdmas[(h + 1, sub)] = start_hop(h + 1, sub)
                d_cw = dot_sub(cw_ref[h, sub, :, :], sub)
                d_ccw = dot_sub(ccw_ref[h, sub, :, :], sub)
                acc_cw = d_cw if acc_cw is None else acc_cw + d_cw
                acc_ccw = d_ccw if acc_ccw is None else acc_ccw + d_ccw
            origin_cw = (my - h) % N_DEV
            origin_ccw = (my + h) % N_DEV
            epilogue_store(acc_cw, origin_cw * m_per, m_half)
            epilogue_store(acc_ccw, origin_ccw * m_per + m_half, m_half)

        for r_cw, r_ccw in rdmas.values():
            r_cw.wait_send()
            r_ccw.wait_send()

    return pl.pallas_call(
        body,
        out_shape=jax.ShapeDtypeStruct((N_DEV * m_per, n), jnp.float32),
        in_specs=[
            pl.BlockSpec(memory_space=pltpu.VMEM),
            pl.BlockSpec(memory_space=pltpu.VMEM),
            pl.BlockSpec(memory_space=pltpu.SMEM),
            pl.BlockSpec(memory_space=pltpu.SMEM),
        ],
        out_specs=pl.BlockSpec(memory_space=pltpu.VMEM),
        scratch_shapes=[
            pltpu.VMEM((N_DEV, N_SUB, m_half, k_sub), jnp.float8_e4m3fn),
            pltpu.VMEM((N_DEV, N_SUB, m_half, k_sub), jnp.float8_e4m3fn),
            pltpu.VMEM((k, n), jnp.bfloat16),
            pltpu.SemaphoreType.DMA((N_DEV - 1, N_SUB)),
            pltpu.SemaphoreType.DMA((N_DEV - 1, N_SUB)),
            pltpu.SemaphoreType.DMA((N_DEV - 1, N_SUB)),
            pltpu.SemaphoreType.DMA((N_DEV - 1, N_SUB)),
        ],
        compiler_params=pltpu.CompilerParams(
            collective_id=0,
            vmem_limit_bytes=100 * 1024 * 1024,
        ),
    )(x, w_mat, scale_x, scale_w)


# device time: 93820 ns/iter; 1.9359x vs baseline; 1.0256x over previous
import jax
import jax.numpy as jnp
from jax import lax
from jax.experimental import pallas as pl
from jax.experimental.pallas import tpu as pltpu

N_DEV = 4
N_SUB = 2


def kernel(x, w_mat, scale_x, scale_w):
    m_per, k = x.shape
    n = w_mat.shape[1]
    m_half = m_per // 2
    k_sub = k // N_SUB

    def body(x_ref, w_ref, sx_ref, sw_ref, out_ref,
             cw_ref, ccw_ref, w8_ref,
             send_cw, recv_cw, send_ccw, recv_ccw):
        my = lax.axis_index("i")
        left = (my - 1) % N_DEV
        right = (my + 1) % N_DEV

        barrier_sem = pltpu.get_barrier_semaphore()
        for nbr in [left, right]:
            pl.semaphore_signal(
                barrier_sem, inc=1,
                device_id=(nbr,), device_id_type=pl.DeviceIdType.MESH,
            )
        pl.semaphore_wait(barrier_sem, 2)

        s = sx_ref[0] * sw_ref[0]

        def start_hop(h, sub):
            r_cw = pltpu.make_async_remote_copy(
                src_ref=cw_ref.at[h - 1, sub], dst_ref=cw_ref.at[h, sub],
                send_sem=send_cw.at[h - 1, sub], recv_sem=recv_cw.at[h - 1, sub],
                device_id=(right,), device_id_type=pl.DeviceIdType.MESH,
            )
            r_ccw = pltpu.make_async_remote_copy(
                src_ref=ccw_ref.at[h - 1, sub], dst_ref=ccw_ref.at[h, sub],
                send_sem=send_ccw.at[h - 1, sub], recv_sem=recv_ccw.at[h - 1, sub],
                device_id=(left,), device_id_type=pl.DeviceIdType.MESH,
            )
            r_cw.start()
            r_ccw.start()
            return r_cw, r_ccw

        rdmas = {}
        for sub in range(N_SUB):
            cols = pl.ds(sub * k_sub, k_sub)
            cw_ref[0, sub, :, :] = x_ref[:m_half, cols].astype(jnp.float8_e4m3fn)
            ccw_ref[0, sub, :, :] = x_ref[m_half:, cols].astype(jnp.float8_e4m3fn)
            rdmas[(1, sub)] = start_hop(1, sub)

        w8_ref[...] = w_ref[...].astype(jnp.float8_e5m2)

        def dot_sub(chunk_ref, sub):
            return jnp.dot(
                chunk_ref, w8_ref[pl.ds(sub * k_sub, k_sub), :],
                preferred_element_type=jnp.float32,
            )

        def accum(ring_ref, h):
            acc = dot_sub(ring_ref[h, 0, :, :], 0)
            for sub in range(1, N_SUB):
                acc = acc + dot_sub(ring_ref[h, sub, :, :], sub)
            return acc

        def epilogue_store(acc, row_start):
            y = acc * s
            z = jnp.clip(y, -60.0, 60.0)
            out_ref[pl.ds(row_start, m_half), :] = y / (1.0 + jnp.exp(-z))

        epilogue_store(accum(cw_ref, 0), my * m_per)
        epilogue_store(accum(ccw_ref, 0), my * m_per + m_half)

        for h in range(1, N_DEV):
            for sub in range(N_SUB):
                r_cw, r_ccw = rdmas[(h, sub)]
                r_cw.wait_recv()
                r_ccw.wait_recv()
                if h + 1 < N_DEV:
                    rdmas[(h + 1, sub)] = start_hop(h + 1, sub)
            origin_cw = (my - h) % N_DEV
            origin_ccw = (my + h) % N_DEV
            epilogue_store(accum(cw_ref, h), origin_cw * m_per)
            epilogue_store(accum(ccw_ref, h), origin_ccw * m_per + m_half)

        for r_cw, r_ccw in rdmas.values():
            r_cw.wait_send()
            r_ccw.wait_send()

    return pl.pallas_call(
        body,
        out_shape=jax.ShapeDtypeStruct((N_DEV * m_per, n), jnp.float32),
        in_specs=[
            pl.BlockSpec(memory_space=pltpu.VMEM),
            pl.BlockSpec(memory_space=pltpu.VMEM),
            pl.BlockSpec(memory_space=pltpu.SMEM),
            pl.BlockSpec(memory_space=pltpu.SMEM),
        ],
        out_specs=pl.BlockSpec(memory_space=pltpu.VMEM),
        scratch_shapes=[
            pltpu.VMEM((N_DEV, N_SUB, m_half, k_sub), jnp.float8_e4m3fn),
            pltpu.VMEM((N_DEV, N_SUB, m_half, k_sub), jnp.float8_e4m3fn),
            pltpu.VMEM((k, n), jnp.float8_e5m2),
            pltpu.SemaphoreType.DMA((N_DEV - 1, N_SUB)),
            pltpu.SemaphoreType.DMA((N_DEV - 1, N_SUB)),
            pltpu.SemaphoreType.DMA((N_DEV - 1, N_SUB)),
            pltpu.SemaphoreType.DMA((N_DEV - 1, N_SUB)),
        ],
        compiler_params=pltpu.CompilerParams(
            collective_id=0,
            vmem_limit_bytes=100 * 1024 * 1024,
        ),
    )(x, w_mat, scale_x, scale_w)
